# baseline (speedup 1.0000x reference)
import functools

import jax
import jax.numpy as jnp
from jax import lax
from jax.experimental import pallas as pl
from jax.experimental.pallas import tpu as pltpu

N_NODES = 50000
N_GRAPHS = 16

E_PAD = 819200
N_PAD = 50176
BE = 2048
BN = 1024
DUMP = N_NODES


def _ured_body(u_ref, wu_ref, bu_ref, out_ref):
    out_ref[...] = (
        jnp.dot(u_ref[...], wu_ref[...], preferred_element_type=jnp.float32)
        + bu_ref[...]
    )


def _edge_body(xr_ref, xc_ref, ured_ref, w0r_ref, w0c_ref, w0u_ref, be0_ref,
               we1_ref, be1_ref, we2_ref, be2_ref, w1c_ref, w1e_ref, bn10_ref,
               wn11_ref, bn11_ref, mlo_ref, mhi_ref):
    f32 = jnp.float32
    xr = xr_ref[...]
    xc = xc_ref[...]
    b = xc[:, 10:11]
    iota = lax.broadcasted_iota(jnp.int32, (1, N_GRAPHS), 1).astype(f32)
    oh = (b == iota).astype(f32)
    ub = jnp.dot(oh, ured_ref[...], preferred_element_type=f32)
    e0 = (jnp.dot(xr, w0r_ref[...], preferred_element_type=f32)
          + jnp.dot(xc, w0c_ref[...], preferred_element_type=f32)
          + jnp.dot(ub, w0u_ref[...], preferred_element_type=f32)
          + be0_ref[...])
    h = jnp.maximum(e0, 0.0)
    h = jnp.maximum(jnp.dot(h, we1_ref[...], preferred_element_type=f32)
                    + be1_ref[...], 0.0)
    eo = jnp.dot(h, we2_ref[...], preferred_element_type=f32) + be2_ref[...]
    m = jnp.maximum(jnp.dot(xc, w1c_ref[...], preferred_element_type=f32)
                    + jnp.dot(eo, w1e_ref[...], preferred_element_type=f32)
                    + bn10_ref[...], 0.0)
    m = jnp.maximum(jnp.dot(m, wn11_ref[...], preferred_element_type=f32)
                    + bn11_ref[...], 0.0)
    mlo_ref[...] = m[:, :32]
    mhi_ref[...] = m[:, 32:]


def _node_body(x_ref, slo_ref, shi_ref, cnt_ref, batch_ref, ured_ref,
               w2x_ref, w2lo_ref, w2hi_ref, w2u_ref, bn20_ref, wn21_ref,
               bn21_ref, out_ref):
    f32 = jnp.float32
    cnt = jnp.sum(cnt_ref[...], axis=1, keepdims=True)
    inv = 1.0 / jnp.maximum(cnt, 1.0)
    slo = slo_ref[...] * inv
    shi = shi_ref[...] * inv
    b = batch_ref[...]
    oh = (b == lax.broadcasted_iota(jnp.int32, (1, N_GRAPHS), 1)).astype(f32)
    ub = jnp.dot(oh, ured_ref[...], preferred_element_type=f32)
    h2 = (jnp.dot(x_ref[...], w2x_ref[...], preferred_element_type=f32)
          + jnp.dot(slo, w2lo_ref[...], preferred_element_type=f32)
          + jnp.dot(shi, w2hi_ref[...], preferred_element_type=f32)
          + jnp.dot(ub, w2u_ref[...], preferred_element_type=f32)
          + bn20_ref[...])
    h2 = jnp.maximum(h2, 0.0)
    out_ref[...] = (jnp.dot(h2, wn21_ref[...], preferred_element_type=f32)
                    + bn21_ref[...])


def _full(shape):
    return pl.BlockSpec(shape, lambda i: (0,) * len(shape))


def kernel(x, edge_index, edge_attr, u, batch, Wu, bu, We0, be0, We1, be1,
           We2, be2, Wn10, bn10, Wn11, bn11, Wn20, bn20, Wn21, bn21):
    f32 = jnp.float32
    row = edge_index[0]
    col = edge_index[1]
    ne = row.shape[0]

    x16 = jnp.pad(x, ((0, N_PAD - N_NODES), (0, 16 - x.shape[1])))
    rowp = jnp.concatenate([row, jnp.full((E_PAD - ne,), DUMP, jnp.int32)])
    colp = jnp.concatenate([col, jnp.zeros((E_PAD - ne,), jnp.int32)])
    eap = jnp.concatenate([edge_attr[:, 0], jnp.zeros((E_PAD - ne,), f32)])
    batchp = jnp.pad(batch, (0, N_PAD - N_NODES))

    z64 = jnp.zeros((16, 64), f32)
    W0r = z64.at[:9].set(We0[0:9])
    W0c = z64.at[:9].set(We0[9:18]).at[9].set(We0[18])
    W0u = We0[19:51]
    W1c = z64.at[:9].set(Wn10[0:9])
    W1e = Wn10[9:73]
    W2x = z64.at[:9].set(Wn20[0:9])
    W2lo = Wn20[9:41]
    W2hi = Wn20[41:73]
    W2u = Wn20[73:105]
    be0r = be0.reshape(1, -1)
    be1r = be1.reshape(1, -1)
    be2r = be2.reshape(1, -1)
    bn10r = bn10.reshape(1, -1)
    bn11r = bn11.reshape(1, -1)
    bn20r = bn20.reshape(1, -1)
    bn21r = bn21.reshape(1, -1)
    bur = bu.reshape(1, -1)

    u_red = pl.pallas_call(
        _ured_body,
        grid=(1,),
        in_specs=[_full((16, 4096)), _full((4096, 32)), _full((1, 32))],
        out_specs=_full((16, 32)),
        out_shape=jax.ShapeDtypeStruct((16, 32), f32),
    )(u, Wu, bur)

    xr = x16[rowp]
    xc0 = x16[colp]
    brow_f = batchp[rowp].astype(f32)
    xc = xc0.at[:, 9].set(eap).at[:, 10].set(brow_f)

    ge = E_PAD // BE
    mlo, mhi = pl.pallas_call(
        _edge_body,
        grid=(ge,),
        in_specs=[
            pl.BlockSpec((BE, 16), lambda i: (i, 0)),
            pl.BlockSpec((BE, 16), lambda i: (i, 0)),
            _full((16, 32)),
            _full((16, 64)), _full((16, 64)), _full((32, 64)), _full((1, 64)),
            _full((64, 64)), _full((1, 64)),
            _full((64, 64)), _full((1, 64)),
            _full((16, 64)), _full((64, 64)), _full((1, 64)),
            _full((64, 64)), _full((1, 64)),
        ],
        out_specs=[
            pl.BlockSpec((BE, 32), lambda i: (i, 0)),
            pl.BlockSpec((BE, 32), lambda i: (i, 0)),
        ],
        out_shape=[
            jax.ShapeDtypeStruct((E_PAD, 32), f32),
            jax.ShapeDtypeStruct((E_PAD, 32), f32),
        ],
    )(xr, xc, u_red, W0r, W0c, W0u, be0r, We1, be1r, We2, be2r,
      W1c, W1e, bn10r, Wn11, bn11r)

    m = jnp.concatenate([mlo, mhi], axis=1)
    seg = jax.ops.segment_sum(m, rowp, num_segments=N_PAD)
    cnt = jax.ops.segment_sum(jnp.ones((E_PAD,), f32), rowp,
                              num_segments=N_PAD)
    seg_lo = seg[:, :32]
    seg_hi = seg[:, 32:]
    cnt16 = jnp.pad(cnt[:, None], ((0, 0), (0, 15)))

    gn = N_PAD // BN
    out = pl.pallas_call(
        _node_body,
        grid=(gn,),
        in_specs=[
            pl.BlockSpec((BN, 16), lambda i: (i, 0)),
            pl.BlockSpec((BN, 32), lambda i: (i, 0)),
            pl.BlockSpec((BN, 32), lambda i: (i, 0)),
            pl.BlockSpec((BN, 16), lambda i: (i, 0)),
            pl.BlockSpec((BN, 1), lambda i: (i, 0)),
            _full((16, 32)),
            _full((16, 64)), _full((32, 64)), _full((32, 64)), _full((32, 64)),
            _full((1, 64)), _full((64, 1)), _full((1, 1)),
        ],
        out_specs=pl.BlockSpec((BN, 1), lambda i: (i, 0)),
        out_shape=jax.ShapeDtypeStruct((N_PAD, 1), f32),
    )(x16, seg_lo, seg_hi, cnt16, batchp[:, None], u_red,
      W2x, W2lo, W2hi, W2u, bn20r, Wn21, bn21r)

    return out[:N_NODES, 0]


# speedup vs baseline: 6.3880x; 6.3880x over previous
import functools

import jax
import jax.numpy as jnp
from jax import lax
from jax.experimental import pallas as pl
from jax.experimental.pallas import tpu as pltpu
from jax.experimental.pallas import tpu_sc as plsc

N_NODES = 50000
N_GRAPHS = 16

E_PAD = 819200
N_PAD = 50176
BE = 2048
BN = 1024
DUMP = N_NODES

SC_K = 1600
EPW_G = E_PAD // 32
EPT_S = E_PAD // 16
NPT = N_PAD // 16
_SC_MESH = dict(core_axis_name="c", subcore_axis_name="s")


def _ured_body(u_ref, wu_ref, bu_ref, out_ref):
    out_ref[...] = (
        jnp.dot(u_ref[...], wu_ref[...], preferred_element_type=jnp.float32)
        + bu_ref[...]
    )


def _edge_body(xr_ref, xc_ref, ea_ref, ured_ref, w0r_ref, w0c_ref, w0u_ref,
               w0e_ref, be0_ref, we1_ref, be1_ref, we2_ref, be2_ref, w1c_ref,
               w1e_ref, bn10_ref, wn11_ref, bn11_ref,
               m0_ref, m1_ref, m2_ref, m3_ref):
    f32 = jnp.float32
    xr = xr_ref[...]
    xc = xc_ref[...]
    b = xr[:, 9:10]
    iota = lax.broadcasted_iota(jnp.int32, (1, N_GRAPHS), 1).astype(f32)
    oh = (b == iota).astype(f32)
    ub = jnp.dot(oh, ured_ref[...], preferred_element_type=f32)
    e0 = (jnp.dot(xr, w0r_ref[...], preferred_element_type=f32)
          + jnp.dot(xc, w0c_ref[...], preferred_element_type=f32)
          + jnp.dot(ub, w0u_ref[...], preferred_element_type=f32)
          + ea_ref[...] * w0e_ref[...]
          + be0_ref[...])
    h = jnp.maximum(e0, 0.0)
    h = jnp.maximum(jnp.dot(h, we1_ref[...], preferred_element_type=f32)
                    + be1_ref[...], 0.0)
    eo = jnp.dot(h, we2_ref[...], preferred_element_type=f32) + be2_ref[...]
    m = jnp.maximum(jnp.dot(xc, w1c_ref[...], preferred_element_type=f32)
                    + jnp.dot(eo, w1e_ref[...], preferred_element_type=f32)
                    + bn10_ref[...], 0.0)
    m = jnp.maximum(jnp.dot(m, wn11_ref[...], preferred_element_type=f32)
                    + bn11_ref[...], 0.0)
    m0_ref[...] = m[:, 0:16]
    m1_ref[...] = m[:, 16:32]
    m2_ref[...] = m[:, 32:48]
    m3_ref[...] = m[:, 48:64]


def _node_body(x_ref, s0_ref, s1_ref, s2_ref, s3_ref, cnt_ref, batch_ref,
               ured_ref, w2x_ref, w2a_ref, w2b_ref, w2c_ref, w2d_ref,
               w2u_ref, bn20_ref, wn21_ref, bn21_ref, out_ref):
    f32 = jnp.float32
    inv = 1.0 / jnp.maximum(cnt_ref[...], 1.0)
    b = batch_ref[...]
    oh = (b == lax.broadcasted_iota(jnp.int32, (1, N_GRAPHS), 1)).astype(f32)
    ub = jnp.dot(oh, ured_ref[...], preferred_element_type=f32)
    h2 = (jnp.dot(x_ref[...], w2x_ref[...], preferred_element_type=f32)
          + jnp.dot(s0_ref[...] * inv, w2a_ref[...], preferred_element_type=f32)
          + jnp.dot(s1_ref[...] * inv, w2b_ref[...], preferred_element_type=f32)
          + jnp.dot(s2_ref[...] * inv, w2c_ref[...], preferred_element_type=f32)
          + jnp.dot(s3_ref[...] * inv, w2d_ref[...], preferred_element_type=f32)
          + jnp.dot(ub, w2u_ref[...], preferred_element_type=f32)
          + bn20_ref[...])
    h2 = jnp.maximum(h2, 0.0)
    out_ref[...] = (jnp.dot(h2, wn21_ref[...], preferred_element_type=f32)
                    + bn21_ref[...])


def _full(shape):
    return pl.BlockSpec(shape, lambda i: (0,) * len(shape))


def _gather_body(x16_hbm, rowp_hbm, colp_hbm, xr_hbm, xcp_hbm,
                 row_v, col_v, xr_v, xc_v, sem1, sem2):
    c = lax.axis_index("c")
    s = lax.axis_index("s")
    wid = s * 2 + c
    base_w = wid * EPW_G

    @pl.loop(0, EPW_G // SC_K)
    def _chunk(it):
        eb = base_w + it * SC_K
        pltpu.sync_copy(rowp_hbm.at[pl.ds(eb, SC_K)], row_v)
        pltpu.sync_copy(colp_hbm.at[pl.ds(eb, SC_K)], col_v)
        cp1 = pltpu.async_copy(x16_hbm.at[row_v], xr_v, sem1)
        cp2 = pltpu.async_copy(x16_hbm.at[col_v], xc_v, sem2)
        cp1.wait()
        cp2.wait()
        pltpu.sync_copy(xr_v, xr_hbm.at[pl.ds(eb, SC_K)])
        pltpu.sync_copy(xc_v, xcp_hbm.at[pl.ds(eb, SC_K)])


def _sc_gather(x16, rowp, colp):
    f32 = jnp.float32
    return pl.kernel(
        _gather_body,
        out_type=[
            jax.ShapeDtypeStruct((E_PAD, 16), f32),
            jax.ShapeDtypeStruct((E_PAD, 16), f32),
        ],
        mesh=plsc.VectorSubcoreMesh(**_SC_MESH),
        scratch_types=[
            pltpu.VMEM((SC_K,), jnp.int32),
            pltpu.VMEM((SC_K,), jnp.int32),
            pltpu.VMEM((SC_K, 16), f32),
            pltpu.VMEM((SC_K, 16), f32),
            pltpu.SemaphoreType.DMA,
            pltpu.SemaphoreType.DMA,
        ],
        compiler_params=pltpu.CompilerParams(use_tc_tiling_on_sc=False),
    )(x16, rowp, colp)


def _scatter_body(m0_hbm, m1_hbm, m2_hbm, m3_hbm, rowp_hbm, z16_hbm, zcol_hbm,
                  ones_hbm, s0_hbm, s1_hbm, s2_hbm, s3_hbm, cnt_hbm,
                  row_v, m_v, ones_v, acc_sh, cnt_sh):
    c = lax.axis_index("c")
    s = lax.axis_index("s")
    r0 = s * NPT
    base_t = s * EPT_S

    for p in range(2):
        pltpu.sync_copy(z16_hbm, acc_sh.at[pl.ds(r0, NPT)])
        if p == 0:
            @pl.when(c == 0)
            def _():
                pltpu.sync_copy(zcol_hbm, cnt_sh.at[pl.ds(r0, NPT)])
                pltpu.sync_copy(ones_hbm, ones_v)

        plsc.subcore_barrier()

        @pl.loop(0, EPT_S // SC_K)
        def _chunk(it):
            eb = base_t + it * SC_K
            pltpu.sync_copy(rowp_hbm.at[pl.ds(eb, SC_K)], row_v)

            @pl.when(c == 0)
            def _():
                pltpu.sync_copy((m0_hbm, m1_hbm)[p].at[pl.ds(eb, SC_K)], m_v)

            @pl.when(c == 1)
            def _():
                pltpu.sync_copy((m2_hbm, m3_hbm)[p].at[pl.ds(eb, SC_K)], m_v)

            pltpu.sync_copy(m_v, acc_sh.at[row_v], add=True)
            if p == 0:
                @pl.when(c == 0)
                def _():
                    pltpu.sync_copy(ones_v, cnt_sh.at[row_v], add=True)

        plsc.subcore_barrier()

        @pl.when(c == 0)
        def _():
            pltpu.sync_copy(acc_sh.at[pl.ds(r0, NPT)],
                            (s0_hbm, s1_hbm)[p].at[pl.ds(r0, NPT)])
            if p == 0:
                pltpu.sync_copy(cnt_sh.at[pl.ds(r0, NPT)],
                                cnt_hbm.at[pl.ds(r0, NPT)])

        @pl.when(c == 1)
        def _():
            pltpu.sync_copy(acc_sh.at[pl.ds(r0, NPT)],
                            (s2_hbm, s3_hbm)[p].at[pl.ds(r0, NPT)])


def _sc_scatter(m0, m1, m2, m3, rowp):
    f32 = jnp.float32
    z16 = jnp.zeros((NPT, 16), f32)
    zcol = jnp.zeros((NPT,), f32)
    ones = jnp.ones((SC_K,), f32)
    return pl.kernel(
        _scatter_body,
        out_type=[
            jax.ShapeDtypeStruct((N_PAD, 16), f32),
            jax.ShapeDtypeStruct((N_PAD, 16), f32),
            jax.ShapeDtypeStruct((N_PAD, 16), f32),
            jax.ShapeDtypeStruct((N_PAD, 16), f32),
            jax.ShapeDtypeStruct((N_PAD,), f32),
        ],
        mesh=plsc.VectorSubcoreMesh(**_SC_MESH),
        scratch_types=[
            pltpu.VMEM((SC_K,), jnp.int32),
            pltpu.VMEM((SC_K, 16), f32),
            pltpu.VMEM((SC_K,), f32),
            pltpu.VMEM_SHARED((N_PAD, 16), f32),
            pltpu.VMEM_SHARED((N_PAD,), f32),
        ],
        compiler_params=pltpu.CompilerParams(use_tc_tiling_on_sc=False),
    )(m0, m1, m2, m3, rowp, z16, zcol, ones)


def kernel(x, edge_index, edge_attr, u, batch, Wu, bu, We0, be0, We1, be1,
           We2, be2, Wn10, bn10, Wn11, bn11, Wn20, bn20, Wn21, bn21):
    f32 = jnp.float32
    row = edge_index[0]
    col = edge_index[1]
    ne = row.shape[0]

    batchp = jnp.pad(batch, (0, N_PAD - N_NODES))
    x16 = jnp.pad(x, ((0, N_PAD - N_NODES), (0, 16 - x.shape[1])))
    x16 = x16.at[:, 9].set(batchp.astype(f32))
    rowp = jnp.concatenate([row, jnp.full((E_PAD - ne,), DUMP, jnp.int32)])
    colp = jnp.concatenate([col, jnp.zeros((E_PAD - ne,), jnp.int32)])
    eap = jnp.concatenate([edge_attr[:, 0], jnp.zeros((E_PAD - ne,), f32)])

    z64 = jnp.zeros((16, 64), f32)
    W0r = z64.at[:9].set(We0[0:9])
    W0c = z64.at[:9].set(We0[9:18])
    W0u = We0[19:51]
    w0e = We0[18:19]
    W1c = z64.at[:9].set(Wn10[0:9])
    W1e = Wn10[9:73]
    W2x = z64.at[:9].set(Wn20[0:9])
    W2a = Wn20[9:25]
    W2b = Wn20[25:41]
    W2c = Wn20[41:57]
    W2d = Wn20[57:73]
    W2u = Wn20[73:105]
    be0r = be0.reshape(1, -1)
    be1r = be1.reshape(1, -1)
    be2r = be2.reshape(1, -1)
    bn10r = bn10.reshape(1, -1)
    bn11r = bn11.reshape(1, -1)
    bn20r = bn20.reshape(1, -1)
    bn21r = bn21.reshape(1, -1)
    bur = bu.reshape(1, -1)

    u_red = pl.pallas_call(
        _ured_body,
        grid=(1,),
        in_specs=[_full((16, 4096)), _full((4096, 32)), _full((1, 32))],
        out_specs=_full((16, 32)),
        out_shape=jax.ShapeDtypeStruct((16, 32), f32),
    )(u, Wu, bur)

    xr, xc = _sc_gather(x16, rowp, colp)

    ge = E_PAD // BE
    m0, m1, m2, m3 = pl.pallas_call(
        _edge_body,
        grid=(ge,),
        in_specs=[
            pl.BlockSpec((BE, 16), lambda i: (i, 0)),
            pl.BlockSpec((BE, 16), lambda i: (i, 0)),
            pl.BlockSpec((BE, 1), lambda i: (i, 0)),
            _full((16, 32)),
            _full((16, 64)), _full((16, 64)), _full((32, 64)), _full((1, 64)),
            _full((1, 64)),
            _full((64, 64)), _full((1, 64)),
            _full((64, 64)), _full((1, 64)),
            _full((16, 64)), _full((64, 64)), _full((1, 64)),
            _full((64, 64)), _full((1, 64)),
        ],
        out_specs=[
            pl.BlockSpec((BE, 16), lambda i: (i, 0)),
            pl.BlockSpec((BE, 16), lambda i: (i, 0)),
            pl.BlockSpec((BE, 16), lambda i: (i, 0)),
            pl.BlockSpec((BE, 16), lambda i: (i, 0)),
        ],
        out_shape=[
            jax.ShapeDtypeStruct((E_PAD, 16), f32),
            jax.ShapeDtypeStruct((E_PAD, 16), f32),
            jax.ShapeDtypeStruct((E_PAD, 16), f32),
            jax.ShapeDtypeStruct((E_PAD, 16), f32),
        ],
    )(xr, xc, eap[:, None], u_red, W0r, W0c, W0u, w0e, be0r, We1, be1r,
      We2, be2r, W1c, W1e, bn10r, Wn11, bn11r)

    s0, s1, s2, s3, cnt = _sc_scatter(m0, m1, m2, m3, rowp)

    gn = N_PAD // BN
    out = pl.pallas_call(
        _node_body,
        grid=(gn,),
        in_specs=[
            pl.BlockSpec((BN, 16), lambda i: (i, 0)),
            pl.BlockSpec((BN, 16), lambda i: (i, 0)),
            pl.BlockSpec((BN, 16), lambda i: (i, 0)),
            pl.BlockSpec((BN, 16), lambda i: (i, 0)),
            pl.BlockSpec((BN, 16), lambda i: (i, 0)),
            pl.BlockSpec((BN, 1), lambda i: (i, 0)),
            pl.BlockSpec((BN, 1), lambda i: (i, 0)),
            _full((16, 32)),
            _full((16, 64)), _full((16, 64)), _full((16, 64)), _full((16, 64)),
            _full((16, 64)), _full((32, 64)),
            _full((1, 64)), _full((64, 1)), _full((1, 1)),
        ],
        out_specs=pl.BlockSpec((BN, 1), lambda i: (i, 0)),
        out_shape=jax.ShapeDtypeStruct((N_PAD, 1), f32),
    )(x16, s0, s1, s2, s3, cnt[:, None], batchp[:, None], u_red,
      W2x, W2a, W2b, W2c, W2d, W2u, bn20r, Wn21, bn21r)

    return out[:N_NODES, 0]
